# baseline (speedup 1.0000x reference)
import functools

import jax
import jax.numpy as jnp
from jax import lax
from jax.experimental import pallas as pl
from jax.experimental.pallas import tpu as pltpu
from jax.experimental.pallas import tpu_sc as plsc

N = 1024
D = 128
DB = 16
E = 131072
H = 64

NC = 2
NS = 16
NW = NC * NS
EPW = E // NW
CB = 128
NCHUNK = EPW // CB
NCK = E // CB

_mesh = plsc.VectorSubcoreMesh(core_axis_name="c", subcore_axis_name="s")
_sc_params = pltpu.CompilerParams(needs_layout_passes=False,
                                  use_tc_tiling_on_sc=False)


def _prep_body(feats_ref, w1_ref, b1_ref, a_ref, b_ref):
    f = feats_ref[...]
    a_ref[...] = (
        jnp.dot(f, w1_ref[:D, :], preferred_element_type=jnp.float32)
        + b1_ref[...]
    )
    b_ref[...] = jnp.dot(f, w1_ref[D:2 * D, :], preferred_element_type=jnp.float32)


_prep = pl.pallas_call(
    _prep_body,
    out_shape=[
        jax.ShapeDtypeStruct((N, H), jnp.float32),
        jax.ShapeDtypeStruct((N, H), jnp.float32),
    ],
)


@functools.partial(
    pl.kernel,
    out_type=jax.ShapeDtypeStruct((E // 8, 128), jnp.float32),
    mesh=_mesh,
    compiler_params=_sc_params,
    scratch_types=[
        pltpu.VMEM((NCHUNK, 2, CB), jnp.int32),
        pltpu.VMEM((DB, CB), jnp.int32),
        pltpu.VMEM((DB, CB), jnp.int32),
        pltpu.VMEM((DB, CB, DB), jnp.float32),
        pltpu.VMEM((DB, CB, DB), jnp.float32),
        pltpu.VMEM((CB // 8, 128), jnp.float32),
        pltpu.VMEM((CB // 8, 128), jnp.float32),
        pltpu.SemaphoreType.DMA,
        pltpu.SemaphoreType.DMA,
        pltpu.SemaphoreType.DMA,
        pltpu.SemaphoreType.DMA,
    ],
)
def _edge_gather(pv_hbm, bb_hbm, bbg_hbm,
                 prs_v, idx0, idx1, rows0, rows1, pk0, pk1,
                 sg0, sg1, sw0, sw1):
    w = lax.axis_index("s") * NC + lax.axis_index("c")
    iota = lax.iota(jnp.int32, 16)
    idxb = [idx0, idx1]
    rows = [rows0, rows1]
    pks = [pk0, pk1]
    sg = [sg0, sg1]
    sw = [sw0, sw1]

    pltpu.sync_copy(pv_hbm.at[pl.ds(w * NCHUNK, NCHUNK)], prs_v)

    def _build_idx(g, par):
        @pl.loop(0, CB // 16)
        def _grp(gg):
            p0v = prs_v[g, 0, pl.ds(gg * 16, 16)]
            p1v = prs_v[g, 1, pl.ds(gg * 16, 16)]
            basev = (p0v * 1024
                     + lax.shift_right_logical(p1v, 7) * 64
                     + (lax.shift_right_logical(p1v, 4) & 7))
            for k in range(DB):
                idxb[par][k, pl.ds(gg * 16, 16)] = (
                    basev + (k >> 3) * 512 + (k & 7) * 8)

    def _issue(g, par):
        for k in range(DB):
            pltpu.async_copy(bb_hbm.at[idxb[par].at[k]], rows[par].at[k],
                             sg[par])

    _build_idx(0, 0)
    _issue(0, 0)
    _build_idx(1, 1)
    _issue(1, 1)

    @pl.loop(0, NCHUNK // 2)
    def _chunk2(g2):
        for par in range(2):
            g = 2 * g2 + par
            base8 = w * (EPW // 8) + g * (CB // 8)
            for k in range(DB):
                pltpu.make_async_copy(bb_hbm.at[idxb[par].at[k]],
                                      rows[par].at[k], sg[par]).wait()
            @pl.when(g2 > 0)
            def _():
                pltpu.make_async_copy(pks[par],
                                      bbg_hbm.at[pl.ds(0, CB // 8)],
                                      sw[par]).wait()

            @pl.loop(0, CB // 16)
            def _grp(gg):
                p1vec = prs_v[g, 1, pl.ds(gg * 16, 16)]
                for k in range(16):
                    e = gg * 16 + k
                    col = p1vec[k] & 15
                    win = plsc.load_gather(
                        rows[par],
                        [iota, jnp.broadcast_to(e, (16,)),
                         jnp.broadcast_to(col, (16,))])
                    pks[par][2 * gg + k // 8, pl.ds((k % 8) * 16, 16)] = win

            pltpu.async_copy(pks[par], bbg_hbm.at[pl.ds(base8, CB // 8)],
                             sw[par])
            @pl.when(g2 < NCHUNK // 2 - 1)
            def _():
                _build_idx(g + 2, par)
                _issue(g + 2, par)

    for par in range(2):
        pltpu.make_async_copy(pks[par], bbg_hbm.at[pl.ds(0, CB // 8)],
                              sw[par]).wait()


def _bbmat_body(bbg_ref, w1c_ref, y0_ref, y1_ref, y2_ref, y3_ref):
    w1c = w1c_ref[...]
    z = jnp.zeros((DB, H), jnp.float32)
    blk = jnp.concatenate(
        [jnp.concatenate([w1c, z], axis=1),
         jnp.concatenate([z, w1c], axis=1)], axis=0)
    bbg = bbg_ref[...]
    for s, y_ref in enumerate([y0_ref, y1_ref, y2_ref, y3_ref]):
        y_ref[...] = jnp.dot(bbg[:, 32 * s:32 * s + 32], blk,
                             preferred_element_type=jnp.float32)


_GB = 16

_bbmat = pl.pallas_call(
    _bbmat_body,
    grid=(_GB,),
    in_specs=[
        pl.BlockSpec((E // 8 // _GB, 128), lambda i: (i, 0)),
        pl.BlockSpec((DB, H), lambda i: (0, 0)),
    ],
    out_specs=[pl.BlockSpec((E // 8 // _GB, 128), lambda i: (i, 0))] * 4,
    out_shape=[jax.ShapeDtypeStruct((E // 8, 128), jnp.float32)] * 4,
)


@functools.partial(
    pl.kernel,
    out_type=[
        jax.ShapeDtypeStruct((NW * N * H // 128, 128), jnp.float32),
        jax.ShapeDtypeStruct((NW, N), jnp.float32),
    ],
    mesh=_mesh,
    compiler_params=_sc_params,
    scratch_types=[
        pltpu.VMEM((NCHUNK, 2, CB), jnp.int32),
        pltpu.VMEM((2 * CB, H), jnp.float32),
        pltpu.VMEM((2 * CB, H), jnp.float32),
        pltpu.VMEM((2 * 4, CB // 8, 128), jnp.float32),
        pltpu.VMEM((N * H // 128, 128), jnp.float32),
        pltpu.VMEM((N,), jnp.float32),
        pltpu.SemaphoreType.DMA,
        pltpu.SemaphoreType.DMA,
    ],
)
def _edge_main(a_hbm, b_hbm, y0_hbm, y1_hbm, y2_hbm, y3_hbm,
               pv_hbm, hs_hbm, cnt_hbm,
               prs_v, ga_v, gb_v, yb_v, hsum_v, cnt_v,
               sem0, sem1):
    w = lax.axis_index("s") * NC + lax.axis_index("c")
    iota = lax.iota(jnp.int32, 16)
    zeros16 = jnp.zeros((16,), jnp.float32)
    ones16 = jnp.full((16,), 1.0, jnp.float32)
    lane0 = iota == 0
    ys_hbm = [y0_hbm, y1_hbm, y2_hbm, y3_hbm]
    sems = [sem0, sem1]

    pltpu.sync_copy(pv_hbm.at[pl.ds(w * NCHUNK, NCHUNK)], prs_v)

    @pl.loop(0, N * H // 128, unroll=4)
    def _zero_h(r):
        for c in range(8):
            hsum_v[r, pl.ds(c * 16, 16)] = zeros16

    @pl.loop(0, N // 16, unroll=16)
    def _zero_c(i):
        cnt_v[pl.ds(i * 16, 16)] = zeros16

    def _issue(g, par):
        pltpu.async_copy(a_hbm.at[prs_v.at[g, 0]],
                         ga_v.at[pl.ds(par * CB, CB)], sems[par])
        pltpu.async_copy(b_hbm.at[prs_v.at[g, 1]],
                         gb_v.at[pl.ds(par * CB, CB)], sems[par])
        for s in range(4):
            pltpu.async_copy(
                ys_hbm[s].at[pl.ds(w * (EPW // 8) + g * (CB // 8), CB // 8)],
                yb_v.at[par * 4 + s], sems[par])

    def _drain(g, par):
        pltpu.make_async_copy(a_hbm.at[prs_v.at[g, 0]],
                              ga_v.at[pl.ds(par * CB, CB)], sems[par]).wait()
        pltpu.make_async_copy(b_hbm.at[prs_v.at[g, 1]],
                              gb_v.at[pl.ds(par * CB, CB)], sems[par]).wait()
        for s in range(4):
            pltpu.make_async_copy(ys_hbm[s].at[pl.ds(0, CB // 8)],
                                  yb_v.at[par * 4 + s], sems[par]).wait()

    _issue(0, 0)
    _issue(1, 1)

    @pl.loop(0, NCHUNK)
    def _chunk(g):
        par = g & 1
        eoff = par * CB
        yoff = par * 4

        @pl.when(par == 0)
        def _():
            _drain(g, 0)

        @pl.when(par == 1)
        def _():
            _drain(g, 1)

        @pl.loop(0, CB // 16)
        def _grp(gg):
            p0vec = prs_v[g, 0, pl.ds(gg * 16, 16)]
            for k in range(16):
                e = gg * 16 + k
                p0s = p0vec[k]
                plsc.addupdate_scatter(
                    cnt_v, [jnp.broadcast_to(p0s, (16,))],
                    ones16, mask=lane0)
                hrow = jnp.broadcast_to(
                    lax.shift_right_logical(p0s, 1), (16,))
                col0 = iota + (p0s & 1) * 64
                s = (k % 8) // 2
                half = k % 2
                q = 2 * gg + k // 8
                for j in range(H // 16):
                    v = (ga_v[eoff + e, pl.ds(j * 16, 16)]
                         + gb_v[eoff + e, pl.ds(j * 16, 16)]
                         + yb_v[yoff + s, q, pl.ds(half * 64 + j * 16, 16)])
                    h16 = jnp.maximum(v, 0.0)
                    plsc.addupdate_scatter(
                        hsum_v, [hrow, col0 + j * 16], h16)

        @pl.when((g < NCHUNK - 2) & (par == 0))
        def _():
            _issue(g + 2, 0)

        @pl.when((g < NCHUNK - 2) & (par == 1))
        def _():
            _issue(g + 2, 1)

    pltpu.sync_copy(hsum_v, hs_hbm.at[pl.ds(w * (N * H // 128), N * H // 128)])
    pltpu.sync_copy(cnt_v, cnt_hbm.at[w])


def _post_body(feats_ref, hs_ref, cnt_ref, w2_ref, b2_ref, out_ref):
    hsp = jnp.sum(hs_ref[...].reshape(NW, N * H // 128, 128), axis=0)
    even = jnp.dot(hsp[:, :H], w2_ref[...],
                   preferred_element_type=jnp.float32)
    odd = jnp.dot(hsp[:, H:], w2_ref[...],
                  preferred_element_type=jnp.float32)
    s = jnp.concatenate([even[:, None, :], odd[:, None, :]],
                        axis=1).reshape(N, D)
    cnt = jnp.sum(cnt_ref[...], axis=0)
    cntc = cnt[:, None]
    out_ref[...] = (feats_ref[...] + s + cntc * b2_ref[...]) / (1.0 + cntc)


_post = pl.pallas_call(
    _post_body,
    out_shape=jax.ShapeDtypeStruct((N, D), jnp.float32),
)


def kernel(object_feats, bboxes_embedding, pairs, W1, b1, W2, b2):
    pv = jnp.transpose(pairs.reshape(NCK, CB, 2), (0, 2, 1))
    bbv = jnp.transpose(
        bboxes_embedding.reshape(N, 8, 128, 2, 8),
        (0, 3, 1, 4, 2)).reshape(N * DB * 64, DB)
    a_tbl, b_tbl = _prep(object_feats, W1, b1.reshape(1, H))
    bbg = _edge_gather(pv, bbv)
    y0, y1, y2, y3 = _bbmat(bbg, W1[2 * D:, :])
    hs, cnt = _edge_main(a_tbl, b_tbl, y0, y1, y2, y3, pv)
    new_feats = _post(object_feats, hs, cnt, W2, b2.reshape(1, D))
    return new_feats, bboxes_embedding, pairs


# speedup vs baseline: 1.0414x; 1.0414x over previous
import functools

import jax
import jax.numpy as jnp
from jax import lax
from jax.experimental import pallas as pl
from jax.experimental.pallas import tpu as pltpu
from jax.experimental.pallas import tpu_sc as plsc

N = 1024
D = 128
DB = 16
E = 131072
H = 64

NC = 2
NS = 16
NW = NC * NS
EPW = E // NW
CB = 128
NCHUNK = EPW // CB
NCK = E // CB

_mesh = plsc.VectorSubcoreMesh(core_axis_name="c", subcore_axis_name="s")
_sc_params = pltpu.CompilerParams(needs_layout_passes=False,
                                  use_tc_tiling_on_sc=False)


def _prep_body(feats_ref, w1_ref, b1_ref, a_ref, b_ref):
    f = feats_ref[...]
    a_ref[...] = (
        jnp.dot(f, w1_ref[:D, :], preferred_element_type=jnp.float32)
        + b1_ref[...]
    )
    b_ref[...] = jnp.dot(f, w1_ref[D:2 * D, :], preferred_element_type=jnp.float32)


_prep = pl.pallas_call(
    _prep_body,
    out_shape=[
        jax.ShapeDtypeStruct((N, H), jnp.float32),
        jax.ShapeDtypeStruct((N, H), jnp.float32),
    ],
)


@functools.partial(
    pl.kernel,
    out_type=jax.ShapeDtypeStruct((E // 8, 128), jnp.float32),
    mesh=_mesh,
    compiler_params=_sc_params,
    scratch_types=[
        pltpu.VMEM((NCHUNK, 2, CB), jnp.int32),
        pltpu.VMEM((DB, CB), jnp.int32),
        pltpu.VMEM((DB, CB), jnp.int32),
        pltpu.VMEM((DB, CB, DB), jnp.float32),
        pltpu.VMEM((DB, CB, DB), jnp.float32),
        pltpu.VMEM((CB // 8, 128), jnp.float32),
        pltpu.VMEM((CB // 8, 128), jnp.float32),
        pltpu.SemaphoreType.DMA,
        pltpu.SemaphoreType.DMA,
        pltpu.SemaphoreType.DMA,
        pltpu.SemaphoreType.DMA,
    ],
)
def _edge_gather(pv_hbm, bb_hbm, bbg_hbm,
                 prs_v, idx0, idx1, rows0, rows1, pk0, pk1,
                 sg0, sg1, sw0, sw1):
    w = lax.axis_index("s") * NC + lax.axis_index("c")
    iota = lax.iota(jnp.int32, 16)
    idxb = [idx0, idx1]
    rows = [rows0, rows1]
    pks = [pk0, pk1]
    sg = [sg0, sg1]
    sw = [sw0, sw1]

    pltpu.sync_copy(pv_hbm.at[pl.ds(w * NCHUNK, NCHUNK)], prs_v)

    def _build_idx(g, par):
        @pl.loop(0, CB // 16)
        def _grp(gg):
            p0v = prs_v[g, 0, pl.ds(gg * 16, 16)]
            p1v = prs_v[g, 1, pl.ds(gg * 16, 16)]
            basev = (p0v * 1024
                     + lax.shift_right_logical(p1v, 7) * 64
                     + (lax.shift_right_logical(p1v, 4) & 7))
            for k in range(DB):
                idxb[par][k, pl.ds(gg * 16, 16)] = (
                    basev + (k >> 3) * 512 + (k & 7) * 8)

    def _issue(g, par):
        for k in range(DB):
            pltpu.async_copy(bb_hbm.at[idxb[par].at[k]], rows[par].at[k],
                             sg[par])

    _build_idx(0, 0)
    _issue(0, 0)
    _build_idx(1, 1)
    _issue(1, 1)

    @pl.loop(0, NCHUNK // 2)
    def _chunk2(g2):
        for par in range(2):
            g = 2 * g2 + par
            base8 = w * (EPW // 8) + g * (CB // 8)
            for k in range(DB):
                pltpu.make_async_copy(bb_hbm.at[idxb[par].at[k]],
                                      rows[par].at[k], sg[par]).wait()
            @pl.when(g2 > 0)
            def _():
                pltpu.make_async_copy(pks[par],
                                      bbg_hbm.at[pl.ds(0, CB // 8)],
                                      sw[par]).wait()

            @pl.loop(0, CB // 16)
            def _grp(gg):
                p1vec = prs_v[g, 1, pl.ds(gg * 16, 16)]
                for k in range(16):
                    e = gg * 16 + k
                    col = p1vec[k] & 15
                    win = plsc.load_gather(
                        rows[par],
                        [iota, jnp.broadcast_to(e, (16,)),
                         jnp.broadcast_to(col, (16,))])
                    pks[par][2 * gg + k // 8, pl.ds((k % 8) * 16, 16)] = win

            pltpu.async_copy(pks[par], bbg_hbm.at[pl.ds(base8, CB // 8)],
                             sw[par])
            @pl.when(g2 < NCHUNK // 2 - 1)
            def _():
                _build_idx(g + 2, par)
                _issue(g + 2, par)

    for par in range(2):
        pltpu.make_async_copy(pks[par], bbg_hbm.at[pl.ds(0, CB // 8)],
                              sw[par]).wait()


def _bbcopy_body(src_ref, dst_ref, tok_ref):
    dst_ref[...] = src_ref[...]
    tok_ref[...] = jnp.zeros((8, 128), jnp.float32)


_bbcopy = pl.pallas_call(
    _bbcopy_body,
    grid=(16,),
    in_specs=[pl.BlockSpec((N * N * DB // 128 // 16, 128), lambda i: (i, 0))],
    out_specs=[
        pl.BlockSpec((N * N * DB // 128 // 16, 128), lambda i: (i, 0)),
        pl.BlockSpec((8, 128), lambda i: (0, 0)),
    ],
    out_shape=[
        jax.ShapeDtypeStruct((N * N * DB // 128, 128), jnp.float32),
        jax.ShapeDtypeStruct((8, 128), jnp.float32),
    ],
)


def _bbmat_body(bbg_ref, w1c_ref, y0_ref, y1_ref, y2_ref, y3_ref):
    w1c = w1c_ref[...]
    z = jnp.zeros((DB, H), jnp.float32)
    blk = jnp.concatenate(
        [jnp.concatenate([w1c, z], axis=1),
         jnp.concatenate([z, w1c], axis=1)], axis=0)
    bbg = bbg_ref[...]
    for s, y_ref in enumerate([y0_ref, y1_ref, y2_ref, y3_ref]):
        y_ref[...] = jnp.dot(bbg[:, 32 * s:32 * s + 32], blk,
                             preferred_element_type=jnp.float32)


_GB = 16

_bbmat = pl.pallas_call(
    _bbmat_body,
    grid=(_GB,),
    in_specs=[
        pl.BlockSpec((E // 8 // _GB, 128), lambda i: (i, 0)),
        pl.BlockSpec((DB, H), lambda i: (0, 0)),
    ],
    out_specs=[pl.BlockSpec((E // 8 // _GB, 128), lambda i: (i, 0))] * 4,
    out_shape=[jax.ShapeDtypeStruct((E // 8, 128), jnp.float32)] * 4,
)


@functools.partial(
    pl.kernel,
    out_type=[
        jax.ShapeDtypeStruct((NW * N * H // 128, 128), jnp.float32),
        jax.ShapeDtypeStruct((NW, N), jnp.float32),
    ],
    mesh=_mesh,
    compiler_params=_sc_params,
    scratch_types=[
        pltpu.VMEM((NCHUNK, 2, CB), jnp.int32),
        pltpu.VMEM((2 * CB, H), jnp.float32),
        pltpu.VMEM((2 * CB, H), jnp.float32),
        pltpu.VMEM((2 * 4, CB // 8, 128), jnp.float32),
        pltpu.VMEM((N * H // 128, 128), jnp.float32),
        pltpu.VMEM((N,), jnp.float32),
        pltpu.SemaphoreType.DMA,
        pltpu.SemaphoreType.DMA,
    ],
)
def _edge_main(a_hbm, b_hbm, y0_hbm, y1_hbm, y2_hbm, y3_hbm,
               pv_hbm, tok_hbm, hs_hbm, cnt_hbm,
               prs_v, ga_v, gb_v, yb_v, hsum_v, cnt_v,
               sem0, sem1):
    del tok_hbm
    w = lax.axis_index("s") * NC + lax.axis_index("c")
    iota = lax.iota(jnp.int32, 16)
    zeros16 = jnp.zeros((16,), jnp.float32)
    ones16 = jnp.full((16,), 1.0, jnp.float32)
    lane0 = iota == 0
    ys_hbm = [y0_hbm, y1_hbm, y2_hbm, y3_hbm]
    sems = [sem0, sem1]

    pltpu.sync_copy(pv_hbm.at[pl.ds(w * NCHUNK, NCHUNK)], prs_v)

    @pl.loop(0, N * H // 128, unroll=4)
    def _zero_h(r):
        for c in range(8):
            hsum_v[r, pl.ds(c * 16, 16)] = zeros16

    @pl.loop(0, N // 16, unroll=16)
    def _zero_c(i):
        cnt_v[pl.ds(i * 16, 16)] = zeros16

    def _issue(g, par):
        pltpu.async_copy(a_hbm.at[prs_v.at[g, 0]],
                         ga_v.at[pl.ds(par * CB, CB)], sems[par])
        pltpu.async_copy(b_hbm.at[prs_v.at[g, 1]],
                         gb_v.at[pl.ds(par * CB, CB)], sems[par])
        for s in range(4):
            pltpu.async_copy(
                ys_hbm[s].at[pl.ds(w * (EPW // 8) + g * (CB // 8), CB // 8)],
                yb_v.at[par * 4 + s], sems[par])

    def _drain(g, par):
        pltpu.make_async_copy(a_hbm.at[prs_v.at[g, 0]],
                              ga_v.at[pl.ds(par * CB, CB)], sems[par]).wait()
        pltpu.make_async_copy(b_hbm.at[prs_v.at[g, 1]],
                              gb_v.at[pl.ds(par * CB, CB)], sems[par]).wait()
        for s in range(4):
            pltpu.make_async_copy(ys_hbm[s].at[pl.ds(0, CB // 8)],
                                  yb_v.at[par * 4 + s], sems[par]).wait()

    _issue(0, 0)
    _issue(1, 1)

    @pl.loop(0, NCHUNK)
    def _chunk(g):
        par = g & 1
        eoff = par * CB
        yoff = par * 4

        @pl.when(par == 0)
        def _():
            _drain(g, 0)

        @pl.when(par == 1)
        def _():
            _drain(g, 1)

        @pl.loop(0, CB // 16)
        def _grp(gg):
            p0vec = prs_v[g, 0, pl.ds(gg * 16, 16)]
            for k in range(16):
                e = gg * 16 + k
                p0s = p0vec[k]
                plsc.addupdate_scatter(
                    cnt_v, [jnp.broadcast_to(p0s, (16,))],
                    ones16, mask=lane0)
                hrow = jnp.broadcast_to(
                    lax.shift_right_logical(p0s, 1), (16,))
                col0 = iota + (p0s & 1) * 64
                s = (k % 8) // 2
                half = k % 2
                q = 2 * gg + k // 8
                for j in range(H // 16):
                    v = (ga_v[eoff + e, pl.ds(j * 16, 16)]
                         + gb_v[eoff + e, pl.ds(j * 16, 16)]
                         + yb_v[yoff + s, q, pl.ds(half * 64 + j * 16, 16)])
                    h16 = jnp.maximum(v, 0.0)
                    plsc.addupdate_scatter(
                        hsum_v, [hrow, col0 + j * 16], h16)

        @pl.when((g < NCHUNK - 2) & (par == 0))
        def _():
            _issue(g + 2, 0)

        @pl.when((g < NCHUNK - 2) & (par == 1))
        def _():
            _issue(g + 2, 1)

    pltpu.sync_copy(hsum_v, hs_hbm.at[pl.ds(w * (N * H // 128), N * H // 128)])
    pltpu.sync_copy(cnt_v, cnt_hbm.at[w])


def _post_body(feats_ref, hs_ref, cnt_ref, w2_ref, b2_ref, out_ref):
    hsp = jnp.sum(hs_ref[...].reshape(NW, N * H // 128, 128), axis=0)
    even = jnp.dot(hsp[:, :H], w2_ref[...],
                   preferred_element_type=jnp.float32)
    odd = jnp.dot(hsp[:, H:], w2_ref[...],
                  preferred_element_type=jnp.float32)
    s = jnp.concatenate([even[:, None, :], odd[:, None, :]],
                        axis=1).reshape(N, D)
    cnt = jnp.sum(cnt_ref[...], axis=0)
    cntc = cnt[:, None]
    out_ref[...] = (feats_ref[...] + s + cntc * b2_ref[...]) / (1.0 + cntc)


_post = pl.pallas_call(
    _post_body,
    out_shape=jax.ShapeDtypeStruct((N, D), jnp.float32),
)


def kernel(object_feats, bboxes_embedding, pairs, W1, b1, W2, b2):
    pv = jnp.transpose(pairs.reshape(NCK, CB, 2), (0, 2, 1))
    bb5 = jnp.transpose(bboxes_embedding.reshape(N, 8, 128, 2, 8),
                        (0, 3, 1, 4, 2))
    bbv = bb5.reshape(N * DB * 64, DB)
    bb_copy, tok = _bbcopy(bb5.reshape(N * N * DB // 128, 128))
    bb_out = jnp.transpose(bb_copy.reshape(N, 2, 8, 8, 128),
                           (0, 2, 4, 1, 3)).reshape(N, N, DB)
    a_tbl, b_tbl = _prep(object_feats, W1, b1.reshape(1, H))
    bbg = _edge_gather(pv, bbv)
    y0, y1, y2, y3 = _bbmat(bbg, W1[2 * D:, :])
    hs, cnt = _edge_main(a_tbl, b_tbl, y0, y1, y2, y3, pv, tok)
    new_feats = _post(object_feats, hs, cnt, W2, b2.reshape(1, D))
    return new_feats, bb_out, pairs


# speedup vs baseline: 1.1066x; 1.0626x over previous
import functools

import jax
import jax.numpy as jnp
from jax import lax
from jax.experimental import pallas as pl
from jax.experimental.pallas import tpu as pltpu
from jax.experimental.pallas import tpu_sc as plsc

N = 1024
D = 128
DB = 16
E = 131072
H = 64

NC = 2
NS = 16
NW = NC * NS
EPW = E // NW
CB = 128
NCHUNK = EPW // CB
NCK = E // CB

_mesh = plsc.VectorSubcoreMesh(core_axis_name="c", subcore_axis_name="s")
_sc_params = pltpu.CompilerParams(needs_layout_passes=False,
                                  use_tc_tiling_on_sc=False)


def _prep_body(feats_ref, w1_ref, b1_ref, a_ref, b_ref):
    f = feats_ref[...]
    a_ref[...] = (
        jnp.dot(f, w1_ref[:D, :], preferred_element_type=jnp.float32)
        + b1_ref[...]
    )
    b_ref[...] = jnp.dot(f, w1_ref[D:2 * D, :], preferred_element_type=jnp.float32)


_prep = pl.pallas_call(
    _prep_body,
    out_shape=[
        jax.ShapeDtypeStruct((N, H), jnp.float32),
        jax.ShapeDtypeStruct((N, H), jnp.float32),
    ],
)


@functools.partial(
    pl.kernel,
    out_type=jax.ShapeDtypeStruct((E // 8, 128), jnp.float32),
    mesh=_mesh,
    compiler_params=_sc_params,
    scratch_types=[
        pltpu.VMEM((NCHUNK, 2, CB), jnp.int32),
        pltpu.VMEM((DB, CB), jnp.int32),
        pltpu.VMEM((DB, CB), jnp.int32),
        pltpu.VMEM((DB, CB, DB), jnp.float32),
        pltpu.VMEM((DB, CB, DB), jnp.float32),
        pltpu.VMEM((CB // 8, 128), jnp.float32),
        pltpu.VMEM((CB // 8, 128), jnp.float32),
        pltpu.SemaphoreType.DMA,
        pltpu.SemaphoreType.DMA,
        pltpu.SemaphoreType.DMA,
        pltpu.SemaphoreType.DMA,
    ],
)
def _edge_gather(pv_hbm, bb_hbm, bbg_hbm,
                 prs_v, idx0, idx1, rows0, rows1, pk0, pk1,
                 sg0, sg1, sw0, sw1):
    w = lax.axis_index("s") * NC + lax.axis_index("c")
    iota = lax.iota(jnp.int32, 16)
    idxb = [idx0, idx1]
    rows = [rows0, rows1]
    pks = [pk0, pk1]
    sg = [sg0, sg1]
    sw = [sw0, sw1]

    pltpu.sync_copy(pv_hbm.at[pl.ds(w * NCHUNK, NCHUNK)], prs_v)

    def _build_idx(g, par):
        @pl.loop(0, CB // 16)
        def _grp(gg):
            p0v = prs_v[g, 0, pl.ds(gg * 16, 16)]
            p1v = prs_v[g, 1, pl.ds(gg * 16, 16)]
            basev = (p0v * 1024
                     + lax.shift_right_logical(p1v, 7) * 64
                     + (lax.shift_right_logical(p1v, 4) & 7))
            for k in range(DB):
                idxb[par][k, pl.ds(gg * 16, 16)] = (
                    basev + (k >> 3) * 512 + (k & 7) * 8)

    def _issue(g, par):
        for k in range(DB):
            pltpu.async_copy(bb_hbm.at[idxb[par].at[k]], rows[par].at[k],
                             sg[par])

    _build_idx(0, 0)
    _issue(0, 0)
    _build_idx(1, 1)
    _issue(1, 1)

    @pl.loop(0, NCHUNK // 2)
    def _chunk2(g2):
        for par in range(2):
            g = 2 * g2 + par
            base8 = w * (EPW // 8) + g * (CB // 8)
            for k in range(DB):
                pltpu.make_async_copy(bb_hbm.at[idxb[par].at[k]],
                                      rows[par].at[k], sg[par]).wait()
            @pl.when(g2 > 0)
            def _():
                pltpu.make_async_copy(pks[par],
                                      bbg_hbm.at[pl.ds(0, CB // 8)],
                                      sw[par]).wait()

            @pl.loop(0, CB // 16)
            def _grp(gg):
                p1vec = prs_v[g, 1, pl.ds(gg * 16, 16)]
                for k in range(16):
                    e = gg * 16 + k
                    col = p1vec[k] & 15
                    win = plsc.load_gather(
                        rows[par],
                        [iota, jnp.broadcast_to(e, (16,)),
                         jnp.broadcast_to(col, (16,))])
                    pks[par][2 * gg + k // 8, pl.ds((k % 8) * 16, 16)] = win

            pltpu.async_copy(pks[par], bbg_hbm.at[pl.ds(base8, CB // 8)],
                             sw[par])
            @pl.when(g2 < NCHUNK // 2 - 1)
            def _():
                _build_idx(g + 2, par)
                _issue(g + 2, par)

    for par in range(2):
        pltpu.make_async_copy(pks[par], bbg_hbm.at[pl.ds(0, CB // 8)],
                              sw[par]).wait()


def _bbcopy_body(src_ref, dep_ref, dst_ref, tok_ref):
    del dep_ref
    dst_ref[...] = src_ref[...]
    tok_ref[...] = jnp.zeros((8, 128), jnp.float32)


_bbcopy = pl.pallas_call(
    _bbcopy_body,
    grid=(16,),
    in_specs=[
        pl.BlockSpec((N * N * DB // 128 // 16, 128), lambda i: (i, 0)),
        pl.BlockSpec((8, 128), lambda i: (0, 0)),
    ],
    out_specs=[
        pl.BlockSpec((N * N * DB // 128 // 16, 128), lambda i: (i, 0)),
        pl.BlockSpec((8, 128), lambda i: (0, 0)),
    ],
    out_shape=[
        jax.ShapeDtypeStruct((N * N * DB // 128, 128), jnp.float32),
        jax.ShapeDtypeStruct((8, 128), jnp.float32),
    ],
)


def _bbmat_body(bbg_ref, w1c_ref, y0_ref, y1_ref, y2_ref, y3_ref):
    w1c = w1c_ref[...]
    z = jnp.zeros((DB, H), jnp.float32)
    blk = jnp.concatenate(
        [jnp.concatenate([w1c, z], axis=1),
         jnp.concatenate([z, w1c], axis=1)], axis=0)
    bbg = bbg_ref[...]
    for s, y_ref in enumerate([y0_ref, y1_ref, y2_ref, y3_ref]):
        y_ref[...] = jnp.dot(bbg[:, 32 * s:32 * s + 32], blk,
                             preferred_element_type=jnp.float32)


_GB = 16

_bbmat = pl.pallas_call(
    _bbmat_body,
    grid=(_GB,),
    in_specs=[
        pl.BlockSpec((E // 8 // _GB, 128), lambda i: (i, 0)),
        pl.BlockSpec((DB, H), lambda i: (0, 0)),
    ],
    out_specs=[pl.BlockSpec((E // 8 // _GB, 128), lambda i: (i, 0))] * 4,
    out_shape=[jax.ShapeDtypeStruct((E // 8, 128), jnp.float32)] * 4,
)


@functools.partial(
    pl.kernel,
    out_type=[
        jax.ShapeDtypeStruct((NW * N * H // 128, 128), jnp.float32),
        jax.ShapeDtypeStruct((NW, N), jnp.float32),
    ],
    mesh=_mesh,
    compiler_params=_sc_params,
    scratch_types=[
        pltpu.VMEM((NCHUNK, 2, CB), jnp.int32),
        pltpu.VMEM((2 * CB, H), jnp.float32),
        pltpu.VMEM((2 * CB, H), jnp.float32),
        pltpu.VMEM((2 * 4, CB // 8, 128), jnp.float32),
        pltpu.VMEM((N * H // 128, 128), jnp.float32),
        pltpu.VMEM((N,), jnp.float32),
        pltpu.SemaphoreType.DMA,
        pltpu.SemaphoreType.DMA,
    ],
)
def _edge_main(a_hbm, b_hbm, y0_hbm, y1_hbm, y2_hbm, y3_hbm,
               pv_hbm, hs_hbm, cnt_hbm,
               prs_v, ga_v, gb_v, yb_v, hsum_v, cnt_v,
               sem0, sem1):
    w = lax.axis_index("s") * NC + lax.axis_index("c")
    iota = lax.iota(jnp.int32, 16)
    zeros16 = jnp.zeros((16,), jnp.float32)
    ones16 = jnp.full((16,), 1.0, jnp.float32)
    lane0 = iota == 0
    ys_hbm = [y0_hbm, y1_hbm, y2_hbm, y3_hbm]
    sems = [sem0, sem1]

    pltpu.sync_copy(pv_hbm.at[pl.ds(w * NCHUNK, NCHUNK)], prs_v)

    @pl.loop(0, N * H // 128, unroll=4)
    def _zero_h(r):
        for c in range(8):
            hsum_v[r, pl.ds(c * 16, 16)] = zeros16

    @pl.loop(0, N // 16, unroll=16)
    def _zero_c(i):
        cnt_v[pl.ds(i * 16, 16)] = zeros16

    def _issue(g, par):
        pltpu.async_copy(a_hbm.at[prs_v.at[g, 0]],
                         ga_v.at[pl.ds(par * CB, CB)], sems[par])
        pltpu.async_copy(b_hbm.at[prs_v.at[g, 1]],
                         gb_v.at[pl.ds(par * CB, CB)], sems[par])
        for s in range(4):
            pltpu.async_copy(
                ys_hbm[s].at[pl.ds(w * (EPW // 8) + g * (CB // 8), CB // 8)],
                yb_v.at[par * 4 + s], sems[par])

    def _drain(g, par):
        pltpu.make_async_copy(a_hbm.at[prs_v.at[g, 0]],
                              ga_v.at[pl.ds(par * CB, CB)], sems[par]).wait()
        pltpu.make_async_copy(b_hbm.at[prs_v.at[g, 1]],
                              gb_v.at[pl.ds(par * CB, CB)], sems[par]).wait()
        for s in range(4):
            pltpu.make_async_copy(ys_hbm[s].at[pl.ds(0, CB // 8)],
                                  yb_v.at[par * 4 + s], sems[par]).wait()

    _issue(0, 0)
    _issue(1, 1)

    @pl.loop(0, NCHUNK)
    def _chunk(g):
        par = g & 1
        eoff = par * CB
        yoff = par * 4

        @pl.when(par == 0)
        def _():
            _drain(g, 0)

        @pl.when(par == 1)
        def _():
            _drain(g, 1)

        @pl.loop(0, CB // 16)
        def _grp(gg):
            p0vec = prs_v[g, 0, pl.ds(gg * 16, 16)]
            for k in range(16):
                e = gg * 16 + k
                p0s = p0vec[k]
                plsc.addupdate_scatter(
                    cnt_v, [jnp.broadcast_to(p0s, (16,))],
                    ones16, mask=lane0)
                hrow = jnp.broadcast_to(
                    lax.shift_right_logical(p0s, 1), (16,))
                col0 = iota + (p0s & 1) * 64
                s = (k % 8) // 2
                half = k % 2
                q = 2 * gg + k // 8
                for j in range(H // 16):
                    v = (ga_v[eoff + e, pl.ds(j * 16, 16)]
                         + gb_v[eoff + e, pl.ds(j * 16, 16)]
                         + yb_v[yoff + s, q, pl.ds(half * 64 + j * 16, 16)])
                    h16 = jnp.maximum(v, 0.0)
                    plsc.addupdate_scatter(
                        hsum_v, [hrow, col0 + j * 16], h16)

        @pl.when((g < NCHUNK - 2) & (par == 0))
        def _():
            _issue(g + 2, 0)

        @pl.when((g < NCHUNK - 2) & (par == 1))
        def _():
            _issue(g + 2, 1)

    pltpu.sync_copy(hsum_v, hs_hbm.at[pl.ds(w * (N * H // 128), N * H // 128)])
    pltpu.sync_copy(cnt_v, cnt_hbm.at[w])


def _post_body(feats_ref, hs_ref, cnt_ref, w2_ref, b2_ref, tok_ref, out_ref):
    del tok_ref
    hsp = jnp.sum(hs_ref[...].reshape(NW, N * H // 128, 128), axis=0)
    even = jnp.dot(hsp[:, :H], w2_ref[...],
                   preferred_element_type=jnp.float32)
    odd = jnp.dot(hsp[:, H:], w2_ref[...],
                  preferred_element_type=jnp.float32)
    s = jnp.concatenate([even[:, None, :], odd[:, None, :]],
                        axis=1).reshape(N, D)
    cnt = jnp.sum(cnt_ref[...], axis=0)
    cntc = cnt[:, None]
    out_ref[...] = (feats_ref[...] + s + cntc * b2_ref[...]) / (1.0 + cntc)


_post = pl.pallas_call(
    _post_body,
    out_shape=jax.ShapeDtypeStruct((N, D), jnp.float32),
)


def kernel(object_feats, bboxes_embedding, pairs, W1, b1, W2, b2):
    pv = jnp.transpose(pairs.reshape(NCK, CB, 2), (0, 2, 1))
    bb5 = jnp.transpose(bboxes_embedding.reshape(N, 8, 128, 2, 8),
                        (0, 3, 1, 4, 2))
    bbv = bb5.reshape(N * DB * 64, DB)
    a_tbl, b_tbl = _prep(object_feats, W1, b1.reshape(1, H))
    bbg = _edge_gather(pv, bbv)
    bb_copy, tok = _bbcopy(bb5.reshape(N * N * DB // 128, 128), bbg[:8])
    bb_out = jnp.transpose(bb_copy.reshape(N, 2, 8, 8, 128),
                           (0, 2, 4, 1, 3)).reshape(N, N, DB)
    y0, y1, y2, y3 = _bbmat(bbg, W1[2 * D:, :])
    hs, cnt = _edge_main(a_tbl, b_tbl, y0, y1, y2, y3, pv)
    new_feats = _post(object_feats, hs, cnt, W2, b2.reshape(1, D), tok)
    return new_feats, bb_out, pairs


# speedup vs baseline: 1.1066x; 1.0001x over previous
import functools

import jax
import jax.numpy as jnp
from jax import lax
from jax.experimental import pallas as pl
from jax.experimental.pallas import tpu as pltpu
from jax.experimental.pallas import tpu_sc as plsc

N = 1024
D = 128
DB = 16
E = 131072
H = 64

NC = 2
NS = 16
NW = NC * NS
EPW = E // NW
CB = 128
NCHUNK = EPW // CB
NCK = E // CB

_mesh = plsc.VectorSubcoreMesh(core_axis_name="c", subcore_axis_name="s")
_sc_params = pltpu.CompilerParams(needs_layout_passes=False,
                                  use_tc_tiling_on_sc=False)


def _prep_body(feats_ref, w1_ref, b1_ref, a_ref, b_ref):
    f = feats_ref[...]
    a_ref[...] = (
        jnp.dot(f, w1_ref[:D, :], preferred_element_type=jnp.float32)
        + b1_ref[...]
    )
    b_ref[...] = jnp.dot(f, w1_ref[D:2 * D, :], preferred_element_type=jnp.float32)


_prep = pl.pallas_call(
    _prep_body,
    out_shape=[
        jax.ShapeDtypeStruct((N, H), jnp.float32),
        jax.ShapeDtypeStruct((N, H), jnp.float32),
    ],
)


@functools.partial(
    pl.kernel,
    out_type=jax.ShapeDtypeStruct((E // 8, 128), jnp.float32),
    mesh=_mesh,
    compiler_params=_sc_params,
    scratch_types=[
        pltpu.VMEM((NCHUNK, 2, CB), jnp.int32),
        pltpu.VMEM((DB, CB), jnp.int32),
        pltpu.VMEM((DB, CB), jnp.int32),
        pltpu.VMEM((DB, CB, DB), jnp.float32),
        pltpu.VMEM((DB, CB, DB), jnp.float32),
        pltpu.VMEM((CB // 8, 128), jnp.float32),
        pltpu.VMEM((CB // 8, 128), jnp.float32),
        pltpu.SemaphoreType.DMA,
        pltpu.SemaphoreType.DMA,
        pltpu.SemaphoreType.DMA,
        pltpu.SemaphoreType.DMA,
    ],
)
def _edge_gather(pv_hbm, bb_hbm, bbg_hbm,
                 prs_v, idx0, idx1, rows0, rows1, pk0, pk1,
                 sg0, sg1, sw0, sw1):
    w = lax.axis_index("s") * NC + lax.axis_index("c")
    iota = lax.iota(jnp.int32, 16)
    idxb = [idx0, idx1]
    rows = [rows0, rows1]
    pks = [pk0, pk1]
    sg = [sg0, sg1]
    sw = [sw0, sw1]

    pltpu.sync_copy(pv_hbm.at[pl.ds(w * NCHUNK, NCHUNK)], prs_v)

    def _build_idx(g, par):
        @pl.loop(0, CB // 16)
        def _grp(gg):
            p0v = prs_v[g, 0, pl.ds(gg * 16, 16)]
            p1v = prs_v[g, 1, pl.ds(gg * 16, 16)]
            basev = (p0v * 1024
                     + lax.shift_right_logical(p1v, 7) * 64
                     + (lax.shift_right_logical(p1v, 4) & 7))
            for k in range(DB):
                idxb[par][k, pl.ds(gg * 16, 16)] = (
                    basev + (k >> 3) * 512 + (k & 7) * 8)

    def _issue(g, par):
        for k in range(DB):
            pltpu.async_copy(bb_hbm.at[idxb[par].at[k]], rows[par].at[k],
                             sg[par])

    _build_idx(0, 0)
    _issue(0, 0)
    _build_idx(1, 1)
    _issue(1, 1)

    @pl.loop(0, NCHUNK // 2)
    def _chunk2(g2):
        for par in range(2):
            g = 2 * g2 + par
            base8 = w * (EPW // 8) + g * (CB // 8)
            for k in range(DB):
                pltpu.make_async_copy(bb_hbm.at[idxb[par].at[k]],
                                      rows[par].at[k], sg[par]).wait()
            @pl.when(g2 > 0)
            def _():
                pltpu.make_async_copy(pks[par],
                                      bbg_hbm.at[pl.ds(0, CB // 8)],
                                      sw[par]).wait()

            @pl.loop(0, CB // 16)
            def _grp(gg):
                p1vec = prs_v[g, 1, pl.ds(gg * 16, 16)]
                for k in range(16):
                    e = gg * 16 + k
                    col = p1vec[k] & 15
                    win = plsc.load_gather(
                        rows[par],
                        [iota, jnp.broadcast_to(e, (16,)),
                         jnp.broadcast_to(col, (16,))])
                    pks[par][2 * gg + k // 8, pl.ds((k % 8) * 16, 16)] = win

            pltpu.async_copy(pks[par], bbg_hbm.at[pl.ds(base8, CB // 8)],
                             sw[par])
            @pl.when(g2 < NCHUNK // 2 - 1)
            def _():
                _build_idx(g + 2, par)
                _issue(g + 2, par)

    for par in range(2):
        pltpu.make_async_copy(pks[par], bbg_hbm.at[pl.ds(0, CB // 8)],
                              sw[par]).wait()


def _bbcopy_body(src_ref, dep_ref, dst_ref, tok_ref):
    del dep_ref
    dst_ref[...] = src_ref[...]
    tok_ref[...] = jnp.zeros((8, 128), jnp.float32)


_bbcopy = pl.pallas_call(
    _bbcopy_body,
    grid=(16,),
    in_specs=[
        pl.BlockSpec((N * N * DB // 128 // 16, 128), lambda i: (i, 0)),
        pl.BlockSpec((8, 128), lambda i: (0, 0)),
    ],
    out_specs=[
        pl.BlockSpec((N * N * DB // 128 // 16, 128), lambda i: (i, 0)),
        pl.BlockSpec((8, 128), lambda i: (0, 0)),
    ],
    out_shape=[
        jax.ShapeDtypeStruct((N * N * DB // 128, 128), jnp.float32),
        jax.ShapeDtypeStruct((8, 128), jnp.float32),
    ],
)


def _bbmat_body(bbg_ref, w1c_ref, y0_ref, y1_ref, y2_ref, y3_ref):
    w1c = w1c_ref[...]
    z = jnp.zeros((DB, H), jnp.float32)
    blk = jnp.concatenate(
        [jnp.concatenate([w1c, z], axis=1),
         jnp.concatenate([z, w1c], axis=1)], axis=0)
    bbg = bbg_ref[...]
    for s, y_ref in enumerate([y0_ref, y1_ref, y2_ref, y3_ref]):
        y_ref[...] = jnp.dot(bbg[:, 32 * s:32 * s + 32], blk,
                             preferred_element_type=jnp.float32)


_GB = 16

_bbmat = pl.pallas_call(
    _bbmat_body,
    grid=(_GB,),
    in_specs=[
        pl.BlockSpec((E // 8 // _GB, 128), lambda i: (i, 0)),
        pl.BlockSpec((DB, H), lambda i: (0, 0)),
    ],
    out_specs=[pl.BlockSpec((E // 8 // _GB, 128), lambda i: (i, 0))] * 4,
    out_shape=[jax.ShapeDtypeStruct((E // 8, 128), jnp.float32)] * 4,
)


@functools.partial(
    pl.kernel,
    out_type=[
        jax.ShapeDtypeStruct((NW * N * H // 128, 128), jnp.float32),
        jax.ShapeDtypeStruct((NW, N), jnp.float32),
    ],
    mesh=_mesh,
    compiler_params=_sc_params,
    scratch_types=[
        pltpu.VMEM((NCHUNK, 2, CB), jnp.int32),
        pltpu.VMEM((2 * CB, H), jnp.float32),
        pltpu.VMEM((2 * CB, H), jnp.float32),
        pltpu.VMEM((2 * 4, CB // 8, 128), jnp.float32),
        pltpu.VMEM((N * H // 128, 128), jnp.float32),
        pltpu.VMEM((N,), jnp.float32),
        pltpu.SemaphoreType.DMA,
        pltpu.SemaphoreType.DMA,
    ],
)
def _edge_main(a_hbm, b_hbm, y0_hbm, y1_hbm, y2_hbm, y3_hbm,
               pv_hbm, hs_hbm, cnt_hbm,
               prs_v, ga_v, gb_v, yb_v, hsum_v, cnt_v,
               sem0, sem1):
    w = lax.axis_index("s") * NC + lax.axis_index("c")
    iota = lax.iota(jnp.int32, 16)
    zeros16 = jnp.zeros((16,), jnp.float32)
    ones16 = jnp.full((16,), 1.0, jnp.float32)
    lane0 = iota == 0
    ys_hbm = [y0_hbm, y1_hbm, y2_hbm, y3_hbm]
    sems = [sem0, sem1]

    pltpu.sync_copy(pv_hbm.at[pl.ds(w * NCHUNK, NCHUNK)], prs_v)

    @pl.loop(0, N * H // 128, unroll=4)
    def _zero_h(r):
        for c in range(8):
            hsum_v[r, pl.ds(c * 16, 16)] = zeros16

    @pl.loop(0, N // 16, unroll=16)
    def _zero_c(i):
        cnt_v[pl.ds(i * 16, 16)] = zeros16

    def _issue(g, par):
        pltpu.async_copy(a_hbm.at[prs_v.at[g, 0]],
                         ga_v.at[pl.ds(par * CB, CB)], sems[par])
        pltpu.async_copy(b_hbm.at[prs_v.at[g, 1]],
                         gb_v.at[pl.ds(par * CB, CB)], sems[par])
        for s in range(4):
            pltpu.async_copy(
                ys_hbm[s].at[pl.ds(w * (EPW // 8) + g * (CB // 8), CB // 8)],
                yb_v.at[par * 4 + s], sems[par])

    def _drain(g, par):
        pltpu.make_async_copy(a_hbm.at[prs_v.at[g, 0]],
                              ga_v.at[pl.ds(par * CB, CB)], sems[par]).wait()
        pltpu.make_async_copy(b_hbm.at[prs_v.at[g, 1]],
                              gb_v.at[pl.ds(par * CB, CB)], sems[par]).wait()
        for s in range(4):
            pltpu.make_async_copy(ys_hbm[s].at[pl.ds(0, CB // 8)],
                                  yb_v.at[par * 4 + s], sems[par]).wait()

    _issue(0, 0)
    _issue(1, 1)

    @pl.loop(0, NCHUNK)
    def _chunk(g):
        par = g & 1
        eoff = par * CB
        yoff = par * 4

        @pl.when(par == 0)
        def _():
            _drain(g, 0)

        @pl.when(par == 1)
        def _():
            _drain(g, 1)

        @pl.loop(0, CB // 16, unroll=2)
        def _grp(gg):
            p0vec = prs_v[g, 0, pl.ds(gg * 16, 16)]
            for k in range(16):
                e = gg * 16 + k
                p0s = p0vec[k]
                plsc.addupdate_scatter(
                    cnt_v, [jnp.broadcast_to(p0s, (16,))],
                    ones16, mask=lane0)
                hrow = jnp.broadcast_to(
                    lax.shift_right_logical(p0s, 1), (16,))
                col0 = iota + (p0s & 1) * 64
                s = (k % 8) // 2
                half = k % 2
                q = 2 * gg + k // 8
                for j in range(H // 16):
                    v = (ga_v[eoff + e, pl.ds(j * 16, 16)]
                         + gb_v[eoff + e, pl.ds(j * 16, 16)]
                         + yb_v[yoff + s, q, pl.ds(half * 64 + j * 16, 16)])
                    h16 = jnp.maximum(v, 0.0)
                    plsc.addupdate_scatter(
                        hsum_v, [hrow, col0 + j * 16], h16)

        @pl.when((g < NCHUNK - 2) & (par == 0))
        def _():
            _issue(g + 2, 0)

        @pl.when((g < NCHUNK - 2) & (par == 1))
        def _():
            _issue(g + 2, 1)

    pltpu.sync_copy(hsum_v, hs_hbm.at[pl.ds(w * (N * H // 128), N * H // 128)])
    pltpu.sync_copy(cnt_v, cnt_hbm.at[w])


def _post_body(feats_ref, hs_ref, cnt_ref, w2_ref, b2_ref, tok_ref, out_ref):
    del tok_ref
    hsp = jnp.sum(hs_ref[...].reshape(NW, N * H // 128, 128), axis=0)
    even = jnp.dot(hsp[:, :H], w2_ref[...],
                   preferred_element_type=jnp.float32)
    odd = jnp.dot(hsp[:, H:], w2_ref[...],
                  preferred_element_type=jnp.float32)
    s = jnp.concatenate([even[:, None, :], odd[:, None, :]],
                        axis=1).reshape(N, D)
    cnt = jnp.sum(cnt_ref[...], axis=0)
    cntc = cnt[:, None]
    out_ref[...] = (feats_ref[...] + s + cntc * b2_ref[...]) / (1.0 + cntc)


_post = pl.pallas_call(
    _post_body,
    out_shape=jax.ShapeDtypeStruct((N, D), jnp.float32),
)


def kernel(object_feats, bboxes_embedding, pairs, W1, b1, W2, b2):
    pv = jnp.transpose(pairs.reshape(NCK, CB, 2), (0, 2, 1))
    bb5 = jnp.transpose(bboxes_embedding.reshape(N, 8, 128, 2, 8),
                        (0, 3, 1, 4, 2))
    bbv = bb5.reshape(N * DB * 64, DB)
    a_tbl, b_tbl = _prep(object_feats, W1, b1.reshape(1, H))
    bbg = _edge_gather(pv, bbv)
    bb_copy, tok = _bbcopy(bb5.reshape(N * N * DB // 128, 128), bbg[:8])
    bb_out = jnp.transpose(bb_copy.reshape(N, 2, 8, 8, 128),
                           (0, 2, 4, 1, 3)).reshape(N, N, DB)
    y0, y1, y2, y3 = _bbmat(bbg, W1[2 * D:, :])
    hs, cnt = _edge_main(a_tbl, b_tbl, y0, y1, y2, y3, pv)
    new_feats = _post(object_feats, hs, cnt, W2, b2.reshape(1, D), tok)
    return new_feats, bb_out, pairs


# speedup vs baseline: 1.3007x; 1.1754x over previous
import functools

import jax
import jax.numpy as jnp
from jax import lax
from jax.experimental import pallas as pl
from jax.experimental.pallas import tpu as pltpu
from jax.experimental.pallas import tpu_sc as plsc

N = 1024
D = 128
DB = 16
E = 131072
H = 64

NC = 2
NS = 16
NW = NC * NS
EPW = E // NW
CB = 128
NCHUNK = EPW // CB
NCK = E // CB

_mesh = plsc.VectorSubcoreMesh(core_axis_name="c", subcore_axis_name="s")
_sc_params = pltpu.CompilerParams(needs_layout_passes=False,
                                  use_tc_tiling_on_sc=False)


def _prep_body(feats_ref, w1_ref, b1_ref, a_ref, b_ref):
    f = feats_ref[...]
    a_ref[...] = (
        jnp.dot(f, w1_ref[:D, :], preferred_element_type=jnp.float32)
        + b1_ref[...]
    )
    b_ref[...] = jnp.dot(f, w1_ref[D:2 * D, :], preferred_element_type=jnp.float32)


_prep = pl.pallas_call(
    _prep_body,
    out_shape=[
        jax.ShapeDtypeStruct((N, H), jnp.float32),
        jax.ShapeDtypeStruct((N, H), jnp.float32),
    ],
)


@functools.partial(
    pl.kernel,
    out_type=jax.ShapeDtypeStruct((E // 8, 128), jnp.float32),
    mesh=_mesh,
    compiler_params=_sc_params,
    scratch_types=[
        pltpu.VMEM((NCHUNK, 2, CB), jnp.int32),
        pltpu.VMEM((DB, CB), jnp.int32),
        pltpu.VMEM((DB, CB), jnp.int32),
        pltpu.VMEM((DB, CB, DB), jnp.float32),
        pltpu.VMEM((DB, CB, DB), jnp.float32),
        pltpu.VMEM((CB // 8, 128), jnp.float32),
        pltpu.VMEM((CB // 8, 128), jnp.float32),
        pltpu.SemaphoreType.DMA,
        pltpu.SemaphoreType.DMA,
        pltpu.SemaphoreType.DMA,
        pltpu.SemaphoreType.DMA,
    ],
)
def _edge_gather(pv_hbm, bb_hbm, bbg_hbm,
                 prs_v, idx0, idx1, rows0, rows1, pk0, pk1,
                 sg0, sg1, sw0, sw1):
    w = lax.axis_index("s") * NC + lax.axis_index("c")
    iota = lax.iota(jnp.int32, 16)
    idxb = [idx0, idx1]
    rows = [rows0, rows1]
    pks = [pk0, pk1]
    sg = [sg0, sg1]
    sw = [sw0, sw1]

    pltpu.sync_copy(pv_hbm.at[pl.ds(w * NCHUNK, NCHUNK)], prs_v)

    def _build_idx(g, par):
        @pl.loop(0, CB // 16)
        def _grp(gg):
            p0v = prs_v[g, 0, pl.ds(gg * 16, 16)]
            p1v = prs_v[g, 1, pl.ds(gg * 16, 16)]
            basev = (p0v * 1024
                     + lax.shift_right_logical(p1v, 7) * 64
                     + (lax.shift_right_logical(p1v, 4) & 7))
            for k in range(DB):
                idxb[par][k, pl.ds(gg * 16, 16)] = (
                    basev + (k >> 3) * 512 + (k & 7) * 8)

    def _issue(g, par):
        for k in range(DB):
            pltpu.async_copy(bb_hbm.at[idxb[par].at[k]], rows[par].at[k],
                             sg[par])

    _build_idx(0, 0)
    _issue(0, 0)
    _build_idx(1, 1)
    _issue(1, 1)

    @pl.loop(0, NCHUNK // 2)
    def _chunk2(g2):
        for par in range(2):
            g = 2 * g2 + par
            base8 = w * (EPW // 8) + g * (CB // 8)
            for k in range(DB):
                pltpu.make_async_copy(bb_hbm.at[idxb[par].at[k]],
                                      rows[par].at[k], sg[par]).wait()
            @pl.when(g2 > 0)
            def _():
                pltpu.make_async_copy(pks[par],
                                      bbg_hbm.at[pl.ds(0, CB // 8)],
                                      sw[par]).wait()

            @pl.loop(0, CB // 16)
            def _grp(gg):
                p1vec = prs_v[g, 1, pl.ds(gg * 16, 16)]
                for k in range(16):
                    e = gg * 16 + k
                    col = p1vec[k] & 15
                    win = plsc.load_gather(
                        rows[par],
                        [iota, jnp.broadcast_to(e, (16,)),
                         jnp.broadcast_to(col, (16,))])
                    pks[par][2 * gg + k // 8, pl.ds((k % 8) * 16, 16)] = win

            pltpu.async_copy(pks[par], bbg_hbm.at[pl.ds(base8, CB // 8)],
                             sw[par])
            @pl.when(g2 < NCHUNK // 2 - 1)
            def _():
                _build_idx(g + 2, par)
                _issue(g + 2, par)

    for par in range(2):
        pltpu.make_async_copy(pks[par], bbg_hbm.at[pl.ds(0, CB // 8)],
                              sw[par]).wait()


def _bbcopy_body(src_ref, dep_ref, dst_ref, tok_ref):
    del dep_ref
    dst_ref[...] = src_ref[...]
    tok_ref[...] = jnp.zeros((8, 128), jnp.float32)


_bbcopy = pl.pallas_call(
    _bbcopy_body,
    grid=(16,),
    in_specs=[
        pl.BlockSpec((N * N * DB // 128 // 16, 128), lambda i: (i, 0)),
        pl.BlockSpec((8, 128), lambda i: (0, 0)),
    ],
    out_specs=[
        pl.BlockSpec((N * N * DB // 128 // 16, 128), lambda i: (i, 0)),
        pl.BlockSpec((8, 128), lambda i: (0, 0)),
    ],
    out_shape=[
        jax.ShapeDtypeStruct((N * N * DB // 128, 128), jnp.float32),
        jax.ShapeDtypeStruct((8, 128), jnp.float32),
    ],
)


def _bbmat_body(bbg_ref, w1c_ref, y0_ref, y1_ref, y2_ref, y3_ref):
    w1c = w1c_ref[...]
    z = jnp.zeros((DB, H), jnp.float32)
    blk = jnp.concatenate(
        [jnp.concatenate([w1c, z], axis=1),
         jnp.concatenate([z, w1c], axis=1)], axis=0)
    bbg = bbg_ref[...]
    for s, y_ref in enumerate([y0_ref, y1_ref, y2_ref, y3_ref]):
        y_ref[...] = jnp.dot(bbg[:, 32 * s:32 * s + 32], blk,
                             preferred_element_type=jnp.float32)


_GB = 16

_bbmat = pl.pallas_call(
    _bbmat_body,
    grid=(_GB,),
    in_specs=[
        pl.BlockSpec((E // 8 // _GB, 128), lambda i: (i, 0)),
        pl.BlockSpec((DB, H), lambda i: (0, 0)),
    ],
    out_specs=[pl.BlockSpec((E // 8 // _GB, 128), lambda i: (i, 0))] * 4,
    out_shape=[jax.ShapeDtypeStruct((E // 8, 128), jnp.float32)] * 4,
)


@functools.partial(
    pl.kernel,
    out_type=[
        jax.ShapeDtypeStruct((NW * N * H // 128, 128), jnp.float32),
        jax.ShapeDtypeStruct((NW, N), jnp.float32),
    ],
    mesh=_mesh,
    compiler_params=_sc_params,
    scratch_types=[
        pltpu.VMEM((NCHUNK, 2, CB), jnp.int32),
        pltpu.VMEM((2 * CB, H), jnp.float32),
        pltpu.VMEM((2 * CB, H), jnp.float32),
        pltpu.VMEM((2 * 4, CB // 8, 128), jnp.float32),
        pltpu.VMEM((N * H // 128, 128), jnp.float32),
        pltpu.VMEM((N,), jnp.float32),
        pltpu.SemaphoreType.DMA,
        pltpu.SemaphoreType.DMA,
    ],
)
def _edge_main(a_hbm, b_hbm, y0_hbm, y1_hbm, y2_hbm, y3_hbm,
               pv_hbm, hs_hbm, cnt_hbm,
               prs_v, ga_v, gb_v, yb_v, hsum_v, cnt_v,
               sem0, sem1):
    w = lax.axis_index("s") * NC + lax.axis_index("c")
    iota = lax.iota(jnp.int32, 16)
    zeros16 = jnp.zeros((16,), jnp.float32)
    ones16 = jnp.full((16,), 1.0, jnp.float32)
    lane0 = iota == 0
    ys_hbm = [y0_hbm, y1_hbm, y2_hbm, y3_hbm]
    sems = [sem0, sem1]

    pltpu.sync_copy(pv_hbm.at[pl.ds(w * NCHUNK, NCHUNK)], prs_v)

    @pl.loop(0, N * H // 128, unroll=4)
    def _zero_h(r):
        for c in range(8):
            hsum_v[r, pl.ds(c * 16, 16)] = zeros16

    @pl.loop(0, N // 16, unroll=16)
    def _zero_c(i):
        cnt_v[pl.ds(i * 16, 16)] = zeros16

    def _issue(g, par):
        pltpu.async_copy(a_hbm.at[prs_v.at[g, 0]],
                         ga_v.at[pl.ds(par * CB, CB)], sems[par])
        pltpu.async_copy(b_hbm.at[prs_v.at[g, 1]],
                         gb_v.at[pl.ds(par * CB, CB)], sems[par])
        for s in range(4):
            pltpu.async_copy(
                ys_hbm[s].at[pl.ds(w * (EPW // 8) + g * (CB // 8), CB // 8)],
                yb_v.at[par * 4 + s], sems[par])

    def _drain(g, par):
        pltpu.make_async_copy(a_hbm.at[prs_v.at[g, 0]],
                              ga_v.at[pl.ds(par * CB, CB)], sems[par]).wait()
        pltpu.make_async_copy(b_hbm.at[prs_v.at[g, 1]],
                              gb_v.at[pl.ds(par * CB, CB)], sems[par]).wait()
        for s in range(4):
            pltpu.make_async_copy(ys_hbm[s].at[pl.ds(0, CB // 8)],
                                  yb_v.at[par * 4 + s], sems[par]).wait()

    _issue(0, 0)
    _issue(1, 1)

    @pl.loop(0, NCHUNK)
    def _chunk(g):
        par = g & 1
        eoff = par * CB
        yoff = par * 4

        @pl.when(par == 0)
        def _():
            _drain(g, 0)

        @pl.when(par == 1)
        def _():
            _drain(g, 1)

        @functools.partial(plsc.parallel_loop, 0, CB // 16)
        def _grp(gg):
            p0vec = prs_v[g, 0, pl.ds(gg * 16, 16)]
            for k in range(16):
                e = gg * 16 + k
                p0s = p0vec[k]
                plsc.addupdate_scatter(
                    cnt_v, [jnp.broadcast_to(p0s, (16,))],
                    ones16, mask=lane0)
                hrow = jnp.broadcast_to(
                    lax.shift_right_logical(p0s, 1), (16,))
                col0 = iota + (p0s & 1) * 64
                s = (k % 8) // 2
                half = k % 2
                q = 2 * gg + k // 8
                for j in range(H // 16):
                    v = (ga_v[eoff + e, pl.ds(j * 16, 16)]
                         + gb_v[eoff + e, pl.ds(j * 16, 16)]
                         + yb_v[yoff + s, q, pl.ds(half * 64 + j * 16, 16)])
                    h16 = jnp.maximum(v, 0.0)
                    plsc.addupdate_scatter(
                        hsum_v, [hrow, col0 + j * 16], h16)

        @pl.when((g < NCHUNK - 2) & (par == 0))
        def _():
            _issue(g + 2, 0)

        @pl.when((g < NCHUNK - 2) & (par == 1))
        def _():
            _issue(g + 2, 1)

    pltpu.sync_copy(hsum_v, hs_hbm.at[pl.ds(w * (N * H // 128), N * H // 128)])
    pltpu.sync_copy(cnt_v, cnt_hbm.at[w])


def _post_body(feats_ref, hs_ref, cnt_ref, w2_ref, b2_ref, tok_ref, out_ref):
    del tok_ref
    hsp = jnp.sum(hs_ref[...].reshape(NW, N * H // 128, 128), axis=0)
    even = jnp.dot(hsp[:, :H], w2_ref[...],
                   preferred_element_type=jnp.float32)
    odd = jnp.dot(hsp[:, H:], w2_ref[...],
                  preferred_element_type=jnp.float32)
    s = jnp.concatenate([even[:, None, :], odd[:, None, :]],
                        axis=1).reshape(N, D)
    cnt = jnp.sum(cnt_ref[...], axis=0)
    cntc = cnt[:, None]
    out_ref[...] = (feats_ref[...] + s + cntc * b2_ref[...]) / (1.0 + cntc)


_post = pl.pallas_call(
    _post_body,
    out_shape=jax.ShapeDtypeStruct((N, D), jnp.float32),
)


def kernel(object_feats, bboxes_embedding, pairs, W1, b1, W2, b2):
    pv = jnp.transpose(pairs.reshape(NCK, CB, 2), (0, 2, 1))
    bb5 = jnp.transpose(bboxes_embedding.reshape(N, 8, 128, 2, 8),
                        (0, 3, 1, 4, 2))
    bbv = bb5.reshape(N * DB * 64, DB)
    a_tbl, b_tbl = _prep(object_feats, W1, b1.reshape(1, H))
    bbg = _edge_gather(pv, bbv)
    bb_copy, tok = _bbcopy(bb5.reshape(N * N * DB // 128, 128), bbg[:8])
    bb_out = jnp.transpose(bb_copy.reshape(N, 2, 8, 8, 128),
                           (0, 2, 4, 1, 3)).reshape(N, N, DB)
    y0, y1, y2, y3 = _bbmat(bbg, W1[2 * D:, :])
    hs, cnt = _edge_main(a_tbl, b_tbl, y0, y1, y2, y3, pv)
    new_feats = _post(object_feats, hs, cnt, W2, b2.reshape(1, D), tok)
    return new_feats, bb_out, pairs
